# baseline (speedup 1.0000x reference)
import functools

import jax
import jax.numpy as jnp
from jax import lax
from jax.experimental import pallas as pl
from jax.experimental.pallas import tpu as pltpu
from jax.experimental.pallas import tpu_sc as plsc

D = 64
LANES = 16
NC = 2
NS = 16
NW = NC * NS
NBUF = 6
INFLT = 3
SCALE = 8.0


def _build(n_sent, seq_len):
  assert n_sent % NW == 0
  spw = n_sent // NW
  mesh = plsc.VectorSubcoreMesh(core_axis_name="c", subcore_axis_name="s")

  @functools.partial(
      pl.kernel,
      out_type=jax.ShapeDtypeStruct((n_sent, seq_len, D), jnp.float32),
      mesh=mesh,
      scratch_types=[
          pltpu.VMEM((spw, seq_len), jnp.int32),
          pltpu.VMEM((NBUF, seq_len, D), jnp.float32),
          pltpu.SemaphoreType.DMA,
          pltpu.SemaphoreType.DMA,
      ],
      compiler_params=pltpu.CompilerParams(use_tc_tiling_on_sc=False),
  )
  def emb(x_hbm, table_hbm, out_hbm, idx_v, rows_v, sem_in, sem_out):
    wid = lax.axis_index("s") * NC + lax.axis_index("c")
    sent0 = wid * spw
    pltpu.sync_copy(x_hbm.at[pl.ds(sent0, spw)], idx_v)

    def start_gather(j, b):
      pltpu.make_async_copy(
          table_hbm.at[idx_v.at[j]], rows_v.at[b], sem_in
      ).start()

    def wait_gather():
      pltpu.make_async_copy(
          table_hbm.at[idx_v.at[0]], rows_v.at[0], sem_in
      ).wait()

    def start_write(j, b):
      pltpu.make_async_copy(
          rows_v.at[b], out_hbm.at[sent0 + j], sem_out
      ).start()

    def wait_write():
      pltpu.make_async_copy(
          rows_v.at[0], out_hbm.at[sent0], sem_out
      ).wait()

    def scale(b):
      @plsc.parallel_loop(0, seq_len, unroll=5)
      def _(r):
        for c4 in range(D // LANES):
          sl = pl.ds(c4 * LANES, LANES)
          rows_v[b, r, sl] = rows_v[b, r, sl] * SCALE

    for j in range(INFLT):
      start_gather(j, j)
    for j in range(INFLT):
      wait_gather()
      scale(j)
      start_gather(j + INFLT, j + INFLT)
      start_write(j, j)

    def steady(j, carry):
      b = j % NBUF
      wait_gather()
      wait_write()
      scale(b)
      start_gather(j + INFLT, (j + INFLT) % NBUF)
      start_write(j, b)
      return carry

    lax.fori_loop(INFLT, spw - INFLT, steady, 0)

    for j in range(spw - INFLT, spw):
      b = j % NBUF
      wait_gather()
      scale(b)
      start_write(j, b)

    for _ in range(NBUF):
      wait_write()

  return emb


_EMB = _build(16384, 50)


def kernel(x, table):
  return _EMB(x, table)
